# Initial kernel scaffold; baseline (speedup 1.0000x reference)
import jax
import jax.numpy as jnp
from jax.experimental import pallas as pl


def kernel(x, edge_index, batch, W_td1, b_td1, W_td2, b_td2, W_bu1, b_bu1, W_bu2, b_bu2):
    raise NotImplementedError("write your pallas kernel here")


# speedup vs baseline: 15.1374x; 15.1374x over previous
import functools

import jax
import jax.numpy as jnp
from jax import lax
from jax.experimental import pallas as pl
from jax.experimental.pallas import tpu as pltpu
from jax.experimental.pallas import tpu_sc as plsc

N = 10000
E = 320000
D = 128
G = 64
NP_ = 10240
NS = 16
NC = 2
CH = 128
NCHUNK = (E + NS * CH - 1) // (NS * CH)
EPT = NCHUNK * CH
EP = EPT * NS
RPT = NP_ // NS

_mesh = plsc.VectorSubcoreMesh(core_axis_name="c", subcore_axis_name="s")
_sc_params = pltpu.CompilerParams(needs_layout_passes=False)


def _deg_body(idx_hbm, deg_hbm, idx_v, hist_v, gath_v, out_v, hist_sh):
    c = lax.axis_index("c")
    s = lax.axis_index("s")

    z16 = jnp.zeros((16,), jnp.float32)

    def zero_hist(i, carry):
        hist_v[pl.ds(i * 16, 16)] = z16
        return carry

    lax.fori_loop(0, NP_ // 16, zero_hist, 0)

    pltpu.sync_copy(idx_hbm.at[c, pl.ds(s * EPT, EPT)], idx_v)

    ones16 = jnp.ones((16,), jnp.float32)

    def accum(i, carry):
        ii = idx_v[pl.ds(i * 16, 16)]
        plsc.addupdate_scatter(hist_v, [ii], ones16)
        return carry

    lax.fori_loop(0, EPT // 16, accum, 0)

    pltpu.sync_copy(hist_v, hist_sh.at[s])
    plsc.subcore_barrier()

    for i in range(NS):
        pltpu.sync_copy(hist_sh.at[i, pl.ds(s * RPT, RPT)], gath_v.at[i])

    def red(j, carry):
        t = gath_v[0, pl.ds(j * 16, 16)]
        for i in range(1, NS):
            t = t + gath_v[i, pl.ds(j * 16, 16)]
        out_v[pl.ds(j * 16, 16)] = t
        return carry

    lax.fori_loop(0, RPT // 16, red, 0)
    pltpu.sync_copy(out_v, deg_hbm.at[c, pl.ds(s * RPT, RPT)])


@functools.partial(
    pl.kernel,
    out_type=jax.ShapeDtypeStruct((NC, NP_), jnp.float32),
    mesh=_mesh,
    scratch_types=[
        pltpu.VMEM((EPT,), jnp.int32),
        pltpu.VMEM((NP_,), jnp.float32),
        pltpu.VMEM((NS, RPT), jnp.float32),
        pltpu.VMEM((RPT,), jnp.float32),
        pltpu.VMEM_SHARED((NS, NP_), jnp.float32),
    ],
    compiler_params=_sc_params,
)
def _deg_call(idx_hbm, deg_hbm, idx_v, hist_v, gath_v, out_v, hist_sh):
    _deg_body(idx_hbm, deg_hbm, idx_v, hist_v, gath_v, out_v, hist_sh)


def _prop_body(tab_hbm, gidx_hbm, sidx_hbm, out_hbm,
               gi_v, si_v, rows_v, zero_v, acc_sh, sem):
    c = lax.axis_index("c")
    s = lax.axis_index("s")

    z16 = jnp.zeros((16,), jnp.float32)

    def zrow(i, carry):
        def zcol(k, carry2):
            zero_v[i, pl.ds(k * 16, 16)] = z16
            return carry2
        return lax.fori_loop(0, D // 16, zcol, carry)

    lax.fori_loop(0, CH, zrow, 0)

    for k in range(RPT // CH):
        pltpu.sync_copy(zero_v, acc_sh.at[pl.ds(s * RPT + k * CH, CH)])
    plsc.subcore_barrier()

    def step(j, carry):
        base = s * EPT + j * CH
        pltpu.sync_copy(gidx_hbm.at[c, pl.ds(base, CH)], gi_v)
        pltpu.sync_copy(sidx_hbm.at[c, pl.ds(base, CH)], si_v)
        pltpu.async_copy(tab_hbm.at[gi_v], rows_v, sem).wait()
        pltpu.sync_copy(rows_v, acc_sh.at[si_v], add=True)
        return carry

    lax.fori_loop(0, NCHUNK, step, 0)
    plsc.subcore_barrier()

    pltpu.sync_copy(acc_sh.at[pl.ds(s * RPT, RPT)],
                    out_hbm.at[c, pl.ds(s * RPT, RPT)])


@functools.partial(
    pl.kernel,
    out_type=jax.ShapeDtypeStruct((NC, NP_, D), jnp.float32),
    mesh=_mesh,
    scratch_types=[
        pltpu.VMEM((CH,), jnp.int32),
        pltpu.VMEM((CH,), jnp.int32),
        pltpu.VMEM((CH, D), jnp.float32),
        pltpu.VMEM((CH, D), jnp.float32),
        pltpu.VMEM_SHARED((NP_, D), jnp.float32),
        pltpu.SemaphoreType.DMA,
    ],
    compiler_params=_sc_params,
)
def _prop_call(tab_hbm, gidx_hbm, sidx_hbm, out_hbm,
               gi_v, si_v, rows_v, zero_v, acc_sh, sem):
    _prop_body(tab_hbm, gidx_hbm, sidx_hbm, out_hbm,
               gi_v, si_v, rows_v, zero_v, acc_sh, sem)


def _k1_body(deg_ref, x_ref, w_ref, hs_ref, dis_ref):
    dis = lax.rsqrt(deg_ref[0, 0] + 1.0)
    h = jnp.dot(x_ref[...], w_ref[0], preferred_element_type=jnp.float32)
    hs_ref[0] = h * dis[:, None]
    dis_ref[0, 0] = dis


def _k1(deg, x_pad, w1s):
    return pl.pallas_call(
        _k1_body,
        grid=(NC,),
        in_specs=[
            pl.BlockSpec((1, 1, NP_), lambda c: (c, 0, 0)),
            pl.BlockSpec((NP_, D), lambda c: (0, 0)),
            pl.BlockSpec((1, D, D), lambda c: (c, 0, 0)),
        ],
        out_specs=[
            pl.BlockSpec((1, NP_, D), lambda c: (c, 0, 0)),
            pl.BlockSpec((1, 1, NP_), lambda c: (c, 0, 0)),
        ],
        out_shape=[
            jax.ShapeDtypeStruct((NC, NP_, D), jnp.float32),
            jax.ShapeDtypeStruct((NC, 1, NP_), jnp.float32),
        ],
    )(deg.reshape(NC, 1, NP_), x_pad, w1s)


def _k2_body(acc_ref, hs1_ref, dis_ref, b1_ref, w2_ref, hs2_ref):
    dis = dis_ref[0, 0]
    h2 = jnp.maximum(dis[:, None] * (acc_ref[0] + hs1_ref[0]) + b1_ref[0], 0.0)
    hs2_ref[0] = jnp.dot(h2, w2_ref[0], preferred_element_type=jnp.float32) * dis[:, None]


def _k2(acc1, hs1, dis, b1s, w2s):
    return pl.pallas_call(
        _k2_body,
        grid=(NC,),
        in_specs=[
            pl.BlockSpec((1, NP_, D), lambda c: (c, 0, 0)),
            pl.BlockSpec((1, NP_, D), lambda c: (c, 0, 0)),
            pl.BlockSpec((1, 1, NP_), lambda c: (c, 0, 0)),
            pl.BlockSpec((1, 1, D), lambda c: (c, 0, 0)),
            pl.BlockSpec((1, D, D), lambda c: (c, 0, 0)),
        ],
        out_specs=pl.BlockSpec((1, NP_, D), lambda c: (c, 0, 0)),
        out_shape=jax.ShapeDtypeStruct((NC, NP_, D), jnp.float32),
    )(acc1, hs1, dis, b1s.reshape(NC, 1, D), w2s)


def _k3_body(acc_ref, hs2_ref, dis_ref, b2_ref, batch_ref, out_ref):
    dis = dis_ref[0, 0]
    out2 = dis[:, None] * (acc_ref[0] + hs2_ref[0]) + b2_ref[0]
    b = batch_ref[0]
    gids = lax.broadcasted_iota(jnp.int32, (G, NP_), 0)
    oh = jnp.where(gids == b[None, :], 1.0, 0.0)
    out_ref[0] = jnp.dot(oh, out2, preferred_element_type=jnp.float32)


def _k3(acc2, hs2, dis, b2s, batch_pad):
    return pl.pallas_call(
        _k3_body,
        grid=(NC,),
        in_specs=[
            pl.BlockSpec((1, NP_, D), lambda c: (c, 0, 0)),
            pl.BlockSpec((1, NP_, D), lambda c: (c, 0, 0)),
            pl.BlockSpec((1, 1, NP_), lambda c: (c, 0, 0)),
            pl.BlockSpec((1, 1, D), lambda c: (c, 0, 0)),
            pl.BlockSpec((1, NP_), lambda c: (0, 0)),
        ],
        out_specs=pl.BlockSpec((1, G, D), lambda c: (c, 0, 0)),
        out_shape=jax.ShapeDtypeStruct((NC, G, D), jnp.float32),
    )(acc2, hs2, dis, b2s.reshape(NC, 1, D), batch_pad)


def kernel(x, edge_index, batch, W_td1, b_td1, W_td2, b_td2,
           W_bu1, b_bu1, W_bu2, b_bu2):
    src = edge_index[0]
    dst = edge_index[1]
    pad = EP - E
    i32 = jnp.int32

    padN = jnp.full((pad,), N, dtype=i32)
    pad0 = jnp.zeros((pad,), dtype=i32)

    degidx = jnp.stack([jnp.concatenate([dst, padN]),
                        jnp.concatenate([src, padN])])
    gidx = jnp.stack([jnp.concatenate([src, pad0]),
                      jnp.concatenate([dst + NP_, jnp.full((pad,), NP_, i32)])])
    sidx = jnp.stack([jnp.concatenate([dst, padN]),
                      jnp.concatenate([src, padN])])

    x_pad = jnp.pad(x, ((0, NP_ - N), (0, 0)))
    batch_pad = jnp.pad(batch, (0, NP_ - N), constant_values=G)[None, :]

    w1s = jnp.stack([W_td1, W_bu1])
    b1s = jnp.stack([b_td1, b_bu1])
    w2s = jnp.stack([W_td2, W_bu2])
    b2s = jnp.stack([b_td2, b_bu2])

    deg = _deg_call(degidx)
    hs1, dis = _k1(deg, x_pad, w1s)
    acc1 = _prop_call(hs1.reshape(NC * NP_, D), gidx, sidx)
    hs2 = _k2(acc1, hs1, dis, b1s, w2s)
    acc2 = _prop_call(hs2.reshape(NC * NP_, D), gidx, sidx)
    out = _k3(acc2, hs2, dis, b2s, batch_pad)
    return jnp.concatenate([out[0], out[1]], axis=1)


# speedup vs baseline: 28.4899x; 1.8821x over previous
import functools

import jax
import jax.numpy as jnp
from jax import lax
from jax.experimental import pallas as pl
from jax.experimental.pallas import tpu as pltpu
from jax.experimental.pallas import tpu_sc as plsc

N = 10000
E = 320000
D = 128
G = 64
NP_ = 10240
NS = 16
NC = 2
CH = 128
NCHUNK = (E + NS * CH - 1) // (NS * CH)
EPT = NCHUNK * CH
EP = EPT * NS
RPT = NP_ // NS

_mesh = plsc.VectorSubcoreMesh(core_axis_name="c", subcore_axis_name="s")
_sc_params = pltpu.CompilerParams(needs_layout_passes=False)


def _deg_body(idx_hbm, deg_hbm, idx_v, hist_v, gath_v, out_v, hist_sh):
    c = lax.axis_index("c")
    s = lax.axis_index("s")

    z16 = jnp.zeros((16,), jnp.float32)

    def zero_hist(i, carry):
        hist_v[pl.ds(i * 16, 16)] = z16
        return carry

    lax.fori_loop(0, NP_ // 16, zero_hist, 0)

    pltpu.sync_copy(idx_hbm.at[c, pl.ds(s * EPT, EPT)], idx_v)

    ones16 = jnp.ones((16,), jnp.float32)

    def accum(i, carry):
        ii = idx_v[pl.ds(i * 16, 16)]
        plsc.addupdate_scatter(hist_v, [ii], ones16)
        return carry

    lax.fori_loop(0, EPT // 16, accum, 0)

    pltpu.sync_copy(hist_v, hist_sh.at[s])
    plsc.subcore_barrier()

    for i in range(NS):
        pltpu.sync_copy(hist_sh.at[i, pl.ds(s * RPT, RPT)], gath_v.at[i])

    def red(j, carry):
        t = gath_v[0, pl.ds(j * 16, 16)]
        for i in range(1, NS):
            t = t + gath_v[i, pl.ds(j * 16, 16)]
        out_v[pl.ds(j * 16, 16)] = t
        return carry

    lax.fori_loop(0, RPT // 16, red, 0)
    pltpu.sync_copy(out_v, deg_hbm.at[c, pl.ds(s * RPT, RPT)])


@functools.partial(
    pl.kernel,
    out_type=jax.ShapeDtypeStruct((NC, NP_), jnp.float32),
    mesh=_mesh,
    scratch_types=[
        pltpu.VMEM((EPT,), jnp.int32),
        pltpu.VMEM((NP_,), jnp.float32),
        pltpu.VMEM((NS, RPT), jnp.float32),
        pltpu.VMEM((RPT,), jnp.float32),
        pltpu.VMEM_SHARED((NS, NP_), jnp.float32),
    ],
    compiler_params=_sc_params,
)
def _deg_call(idx_hbm, deg_hbm, idx_v, hist_v, gath_v, out_v, hist_sh):
    _deg_body(idx_hbm, deg_hbm, idx_v, hist_v, gath_v, out_v, hist_sh)


def _idx_load(eidx_hbm, eb, isem, c, s, m):
    return pltpu.make_async_copy(eidx_hbm.at[c, s, m], eb.at[m % 3],
                                 isem.at[m % 3])


def _prop_body(tab_hbm, eidx_hbm, out_hbm, eb, rows_v, acc_sh, isem, gsem):
    c = lax.axis_index("c")
    s = lax.axis_index("s")

    for m in range(3):
        _idx_load(eidx_hbm, eb, isem, c, s, m).start()

    z16 = jnp.zeros((16,), jnp.float32)

    def zrow(i, carry):
        def zcol(k, carry2):
            rows_v[0, i, pl.ds(k * 16, 16)] = z16
            return carry2
        return lax.fori_loop(0, D // 16, zcol, carry)

    lax.fori_loop(0, CH, zrow, 0)

    for k in range(RPT // CH):
        pltpu.sync_copy(rows_v.at[0], acc_sh.at[pl.ds(s * RPT + k * CH, CH)])
    plsc.subcore_barrier()

    _idx_load(eidx_hbm, eb, isem, c, s, 0).wait()
    pltpu.async_copy(tab_hbm.at[eb.at[0, 0]], rows_v.at[0], gsem.at[0])

    def step(j, carry):
        b = j % 2
        m3 = j % 3

        @pl.when(j + 1 < NCHUNK)
        def _():
            _idx_load(eidx_hbm, eb, isem, c, s, j + 1).wait()
            pltpu.async_copy(tab_hbm.at[eb.at[(j + 1) % 3, 0]],
                             rows_v.at[1 - b], gsem.at[1 - b])

        pltpu.make_async_copy(tab_hbm.at[eb.at[m3, 0]], rows_v.at[b],
                              gsem.at[b]).wait()
        pltpu.sync_copy(rows_v.at[b], acc_sh.at[eb.at[m3, 1]], add=True)

        @pl.when(j + 3 < NCHUNK)
        def _():
            _idx_load(eidx_hbm, eb, isem, c, s, j + 3).start()
        return carry

    lax.fori_loop(0, NCHUNK, step, 0)
    plsc.subcore_barrier()

    pltpu.sync_copy(acc_sh.at[pl.ds(s * RPT, RPT)],
                    out_hbm.at[c, pl.ds(s * RPT, RPT)])


@functools.partial(
    pl.kernel,
    out_type=jax.ShapeDtypeStruct((NC, NP_, D), jnp.float32),
    mesh=_mesh,
    scratch_types=[
        pltpu.VMEM((3, 2, CH), jnp.int32),
        pltpu.VMEM((2, CH, D), jnp.float32),
        pltpu.VMEM_SHARED((NP_, D), jnp.float32),
        pltpu.SemaphoreType.DMA((3,)),
        pltpu.SemaphoreType.DMA((2,)),
    ],
    compiler_params=_sc_params,
)
def _prop_call(tab_hbm, eidx_hbm, out_hbm, eb, rows_v, acc_sh, isem, gsem):
    _prop_body(tab_hbm, eidx_hbm, out_hbm, eb, rows_v, acc_sh, isem, gsem)


def _k1_body(deg_ref, x_ref, w_ref, hs_ref, dis_ref):
    dis = lax.rsqrt(deg_ref[0, 0] + 1.0)
    h = jnp.dot(x_ref[...], w_ref[0], preferred_element_type=jnp.float32)
    hs_ref[0] = h * dis[:, None]
    dis_ref[0, 0] = dis


def _k1(deg, x_pad, w1s):
    return pl.pallas_call(
        _k1_body,
        grid=(NC,),
        in_specs=[
            pl.BlockSpec((1, 1, NP_), lambda c: (c, 0, 0)),
            pl.BlockSpec((NP_, D), lambda c: (0, 0)),
            pl.BlockSpec((1, D, D), lambda c: (c, 0, 0)),
        ],
        out_specs=[
            pl.BlockSpec((1, NP_, D), lambda c: (c, 0, 0)),
            pl.BlockSpec((1, 1, NP_), lambda c: (c, 0, 0)),
        ],
        out_shape=[
            jax.ShapeDtypeStruct((NC, NP_, D), jnp.float32),
            jax.ShapeDtypeStruct((NC, 1, NP_), jnp.float32),
        ],
    )(deg.reshape(NC, 1, NP_), x_pad, w1s)


def _k2_body(acc_ref, hs1_ref, dis_ref, b1_ref, w2_ref, hs2_ref):
    dis = dis_ref[0, 0]
    h2 = jnp.maximum(dis[:, None] * (acc_ref[0] + hs1_ref[0]) + b1_ref[0], 0.0)
    hs2_ref[0] = jnp.dot(h2, w2_ref[0], preferred_element_type=jnp.float32) * dis[:, None]


def _k2(acc1, hs1, dis, b1s, w2s):
    return pl.pallas_call(
        _k2_body,
        grid=(NC,),
        in_specs=[
            pl.BlockSpec((1, NP_, D), lambda c: (c, 0, 0)),
            pl.BlockSpec((1, NP_, D), lambda c: (c, 0, 0)),
            pl.BlockSpec((1, 1, NP_), lambda c: (c, 0, 0)),
            pl.BlockSpec((1, 1, D), lambda c: (c, 0, 0)),
            pl.BlockSpec((1, D, D), lambda c: (c, 0, 0)),
        ],
        out_specs=pl.BlockSpec((1, NP_, D), lambda c: (c, 0, 0)),
        out_shape=jax.ShapeDtypeStruct((NC, NP_, D), jnp.float32),
    )(acc1, hs1, dis, b1s.reshape(NC, 1, D), w2s)


def _k3_body(acc_ref, hs2_ref, dis_ref, b2_ref, batch_ref, out_ref):
    dis = dis_ref[0, 0]
    out2 = dis[:, None] * (acc_ref[0] + hs2_ref[0]) + b2_ref[0]
    b = batch_ref[0]
    gids = lax.broadcasted_iota(jnp.int32, (G, NP_), 0)
    oh = jnp.where(gids == b[None, :], 1.0, 0.0)
    out_ref[0] = jnp.dot(oh, out2, preferred_element_type=jnp.float32)


def _k3(acc2, hs2, dis, b2s, batch_pad):
    return pl.pallas_call(
        _k3_body,
        grid=(NC,),
        in_specs=[
            pl.BlockSpec((1, NP_, D), lambda c: (c, 0, 0)),
            pl.BlockSpec((1, NP_, D), lambda c: (c, 0, 0)),
            pl.BlockSpec((1, 1, NP_), lambda c: (c, 0, 0)),
            pl.BlockSpec((1, 1, D), lambda c: (c, 0, 0)),
            pl.BlockSpec((1, NP_), lambda c: (0, 0)),
        ],
        out_specs=pl.BlockSpec((1, G, D), lambda c: (c, 0, 0)),
        out_shape=jax.ShapeDtypeStruct((NC, G, D), jnp.float32),
    )(acc2, hs2, dis, b2s.reshape(NC, 1, D), batch_pad)


def kernel(x, edge_index, batch, W_td1, b_td1, W_td2, b_td2,
           W_bu1, b_bu1, W_bu2, b_bu2):
    src = edge_index[0]
    dst = edge_index[1]
    pad = EP - E
    i32 = jnp.int32

    padN = jnp.full((pad,), N, dtype=i32)
    pad0 = jnp.zeros((pad,), dtype=i32)

    degidx = jnp.stack([jnp.concatenate([dst, padN]),
                        jnp.concatenate([src, padN])])
    gidx = jnp.stack([jnp.concatenate([src, pad0]),
                      jnp.concatenate([dst + NP_, jnp.full((pad,), NP_, i32)])])
    sidx = jnp.stack([jnp.concatenate([dst, padN]),
                      jnp.concatenate([src, padN])])

    x_pad = jnp.pad(x, ((0, NP_ - N), (0, 0)))
    batch_pad = jnp.pad(batch, (0, NP_ - N), constant_values=G)[None, :]

    w1s = jnp.stack([W_td1, W_bu1])
    b1s = jnp.stack([b_td1, b_bu1])
    w2s = jnp.stack([W_td2, W_bu2])
    b2s = jnp.stack([b_td2, b_bu2])

    deg = _deg_call(degidx)
    hs1, dis = _k1(deg, x_pad, w1s)
    eidx = jnp.stack([gidx.reshape(NC, NS, NCHUNK, CH),
                      sidx.reshape(NC, NS, NCHUNK, CH)], axis=3)
    acc1 = _prop_call(hs1.reshape(NC * NP_, D), eidx)
    hs2 = _k2(acc1, hs1, dis, b1s, w2s)
    acc2 = _prop_call(hs2.reshape(NC * NP_, D), eidx)
    out = _k3(acc2, hs2, dis, b2s, batch_pad)
    return jnp.concatenate([out[0], out[1]], axis=1)
